# Initial kernel scaffold; baseline (speedup 1.0000x reference)
import jax
import jax.numpy as jnp
from jax.experimental import pallas as pl


def kernel(pos, batch, q_A, mu_A, m_A, v_A, E_pred):
    raise NotImplementedError("write your pallas kernel here")


# speedup vs baseline: 20.6501x; 20.6501x over previous
import jax
import jax.numpy as jnp
from jax.experimental import pallas as pl

_COEF = 6414.135151


def _sel(S, c):
    s = jax.lax.broadcasted_iota(jnp.int32, (S, 3 * S), 0)
    j = jax.lax.broadcasted_iota(jnp.int32, (S, 3 * S), 1)
    return jnp.where(j == 3 * s + c, 1.0, 0.0).astype(jnp.float32)


def _proj(x, c):
    S = x.shape[1]
    return jax.lax.dot_general(x, _sel(S, c), (((1,), (0,)), ((), ())),
                               preferred_element_type=jnp.float32)


def _ext(x, c):
    S = x.shape[1] // 3
    return jax.lax.dot_general(x, _sel(S, c), (((1,), (1,)), ((), ())),
                               preferred_element_type=jnp.float32)


def _agg_kernel(ids_ref, pos_ref, q_ref, mu_ref, m_ref, v_ref, ep_ref,
                acc_ref, mu_out_ref, m_out_ref, r_out_ref):
    i = pl.program_id(0)
    Bseg = acc_ref.shape[0]
    K = pos_ref.shape[0]

    ids = ids_ref[...]
    pos = pos_ref[...]
    q = q_ref[...]
    mu = mu_ref[...]
    m = m_ref[...]
    v = v_ref[...]

    px, py, pz = pos[:, 0:1], pos[:, 1:2], pos[:, 2:3]
    vx, vy, vz = _ext(v, 0), _ext(v, 1), _ext(v, 2)
    cx = py * vz - pz * vy
    cy = pz * vx - px * vz
    cz = px * vy - py * vx

    mu_feat = mu + _proj(q * px, 0) + _proj(q * py, 1) + _proj(q * pz, 2)
    m_feat = m + 0.5 * (_proj(cx, 0) + _proj(cy, 1) + _proj(cz, 2))
    ones = jnp.ones((K, 1), jnp.float32)
    feats = jnp.concatenate([mu_feat, m_feat, v, q, pos, ones], axis=1)

    cols = jax.lax.broadcasted_iota(jnp.int32, (K, Bseg), 1)
    onehot = (ids == cols).astype(jnp.float32)
    part = jax.lax.dot_general(onehot, feats, (((0,), (0,)), ((), ())),
                               preferred_element_type=jnp.float32)

    @pl.when(i == 0)
    def _init():
        acc_ref[...] = jnp.zeros_like(acc_ref)

    acc_ref[...] += part

    @pl.when(i == pl.num_programs(0) - 1)
    def _combine():
        acc = acc_ref[...]
        S3 = mu_ref.shape[1]
        S = S3 // 3
        Smu = acc[:, 0:S3]
        Sm = acc[:, S3:2 * S3]
        Sv = acc[:, 2 * S3:3 * S3]
        Sq = acc[:, 3 * S3:3 * S3 + S]
        Sp = acc[:, 3 * S3 + S:3 * S3 + S + 3]
        cnt = acc[:, 3 * S3 + S + 3:3 * S3 + S + 4]
        inv = 1.0 / jnp.maximum(cnt, 1.0)
        pmx, pmy, pmz = Sp[:, 0:1] * inv, Sp[:, 1:2] * inv, Sp[:, 2:3] * inv

        mu_tot = Smu - (_proj(Sq * pmx, 0) + _proj(Sq * pmy, 1)
                        + _proj(Sq * pmz, 2))
        svx, svy, svz = _ext(Sv, 0), _ext(Sv, 1), _ext(Sv, 2)
        ccx = pmy * svz - pmz * svy
        ccy = pmz * svx - pmx * svz
        ccz = pmx * svy - pmy * svx
        m_tot = Sm - 0.5 * (_proj(ccx, 0) + _proj(ccy, 1) + _proj(ccz, 2))

        mu_out_ref[...] = mu_tot
        m_out_ref[...] = m_tot

        g_j = jax.lax.broadcasted_iota(jnp.int32, (S3, S), 0)
        g_s = jax.lax.broadcasted_iota(jnp.int32, (S3, S), 1)
        G = jnp.where(g_j == 3 * g_s + 0, 1.0, 0.0) \
            + jnp.where(g_j == 3 * g_s + 1, 1.0, 0.0) \
            + jnp.where(g_j == 3 * g_s + 2, 1.0, 0.0)
        d = jax.lax.dot_general(mu_tot * m_tot, G.astype(jnp.float32),
                                (((1,), (0,)), ((), ())),
                                preferred_element_type=jnp.float32)
        r_out_ref[...] = _COEF * d / jnp.maximum(ep_ref[...], 1.0)


def kernel(pos, batch, q_A, mu_A, m_A, v_A, E_pred):
    N = pos.shape[0]
    B, S = E_pred.shape
    S3 = 3 * S
    F = 3 * S3 + S + 4

    K = 1000 if N % 1000 == 0 else (500 if N % 500 == 0 else N)
    grid = N // K

    ids = batch.astype(jnp.int32).reshape(N, 1)
    mu_f = mu_A.reshape(N, S3)
    m_f = m_A.reshape(N, S3)
    v_f = v_A.reshape(N, S3)

    f32 = jnp.float32
    acc, mu_flat, m_flat, r = pl.pallas_call(
        _agg_kernel,
        grid=(grid,),
        in_specs=[
            pl.BlockSpec((K, 1), lambda i: (i, 0)),
            pl.BlockSpec((K, 3), lambda i: (i, 0)),
            pl.BlockSpec((K, S), lambda i: (i, 0)),
            pl.BlockSpec((K, S3), lambda i: (i, 0)),
            pl.BlockSpec((K, S3), lambda i: (i, 0)),
            pl.BlockSpec((K, S3), lambda i: (i, 0)),
            pl.BlockSpec((B, S), lambda i: (0, 0)),
        ],
        out_specs=[
            pl.BlockSpec((B, F), lambda i: (0, 0)),
            pl.BlockSpec((B, S3), lambda i: (0, 0)),
            pl.BlockSpec((B, S3), lambda i: (0, 0)),
            pl.BlockSpec((B, S), lambda i: (0, 0)),
        ],
        out_shape=[
            jax.ShapeDtypeStruct((B, F), f32),
            jax.ShapeDtypeStruct((B, S3), f32),
            jax.ShapeDtypeStruct((B, S3), f32),
            jax.ShapeDtypeStruct((B, S), f32),
        ],
    )(ids, pos, q_A, mu_f, m_f, v_f, E_pred)
    del acc
    return (mu_flat.reshape(B, S, 3), m_flat.reshape(B, S, 3), r)


# speedup vs baseline: 23.0948x; 1.1184x over previous
import jax
import jax.numpy as jnp
from jax.experimental import pallas as pl

_COEF = 6414.135151


def _sel(S, c):
    s = jax.lax.broadcasted_iota(jnp.int32, (S, 3 * S), 0)
    j = jax.lax.broadcasted_iota(jnp.int32, (S, 3 * S), 1)
    return jnp.where(j == 3 * s + c, 1.0, 0.0).astype(jnp.float32)


def _proj(x, c):
    S = x.shape[1]
    return jax.lax.dot_general(x, _sel(S, c), (((1,), (0,)), ((), ())),
                               preferred_element_type=jnp.float32)


def _ext(x, c):
    S = x.shape[1] // 3
    return jax.lax.dot_general(x, _sel(S, c), (((1,), (1,)), ((), ())),
                               preferred_element_type=jnp.float32)


def _agg_kernel(ids_ref, pos_ref, q_ref, mu_ref, m_ref, v_ref, ep_ref,
                acc_ref, mu_out_ref, m_out_ref, r_out_ref):
    i = pl.program_id(0)
    Bseg = acc_ref.shape[0]
    K = pos_ref.shape[0]

    ids = ids_ref[...]
    pos = pos_ref[...]
    q = q_ref[...]
    mu = mu_ref[...]
    m = m_ref[...]
    v = v_ref[...]

    px, py, pz = pos[:, 0:1], pos[:, 1:2], pos[:, 2:3]
    vx, vy, vz = _ext(v, 0), _ext(v, 1), _ext(v, 2)
    cx = py * vz - pz * vy
    cy = pz * vx - px * vz
    cz = px * vy - py * vx

    mu_feat = mu + _proj(q * px, 0) + _proj(q * py, 1) + _proj(q * pz, 2)
    m_feat = m + 0.5 * (_proj(cx, 0) + _proj(cy, 1) + _proj(cz, 2))
    ones = jnp.ones((K, 1), jnp.float32)
    feats = jnp.concatenate([mu_feat, m_feat, v, q, pos, ones], axis=1)

    @pl.when(i == 0)
    def _init():
        acc_ref[...] = jnp.zeros_like(acc_ref)

    W = 256
    cols = jax.lax.broadcasted_iota(jnp.int32, (K, W), 1)
    w_lo = jnp.min(ids) // W
    w_hi = jnp.max(ids) // W

    def _window(w, carry):
        start = w * W
        onehot = (ids == cols + start).astype(jnp.float32)
        part = jax.lax.dot_general(onehot, feats, (((0,), (0,)), ((), ())),
                                   preferred_element_type=jnp.float32)
        acc_ref[pl.ds(start, W), :] = acc_ref[pl.ds(start, W), :] + part
        return carry

    jax.lax.fori_loop(w_lo, w_hi + 1, _window, 0)

    @pl.when(i == pl.num_programs(0) - 1)
    def _combine():
        acc = acc_ref[...]
        S3 = mu_ref.shape[1]
        S = S3 // 3
        Smu = acc[:, 0:S3]
        Sm = acc[:, S3:2 * S3]
        Sv = acc[:, 2 * S3:3 * S3]
        Sq = acc[:, 3 * S3:3 * S3 + S]
        Sp = acc[:, 3 * S3 + S:3 * S3 + S + 3]
        cnt = acc[:, 3 * S3 + S + 3:3 * S3 + S + 4]
        inv = 1.0 / jnp.maximum(cnt, 1.0)
        pmx, pmy, pmz = Sp[:, 0:1] * inv, Sp[:, 1:2] * inv, Sp[:, 2:3] * inv

        mu_tot = Smu - (_proj(Sq * pmx, 0) + _proj(Sq * pmy, 1)
                        + _proj(Sq * pmz, 2))
        svx, svy, svz = _ext(Sv, 0), _ext(Sv, 1), _ext(Sv, 2)
        ccx = pmy * svz - pmz * svy
        ccy = pmz * svx - pmx * svz
        ccz = pmx * svy - pmy * svx
        m_tot = Sm - 0.5 * (_proj(ccx, 0) + _proj(ccy, 1) + _proj(ccz, 2))

        mu_out_ref[...] = mu_tot
        m_out_ref[...] = m_tot

        g_j = jax.lax.broadcasted_iota(jnp.int32, (S3, S), 0)
        g_s = jax.lax.broadcasted_iota(jnp.int32, (S3, S), 1)
        G = jnp.where(g_j == 3 * g_s + 0, 1.0, 0.0) \
            + jnp.where(g_j == 3 * g_s + 1, 1.0, 0.0) \
            + jnp.where(g_j == 3 * g_s + 2, 1.0, 0.0)
        d = jax.lax.dot_general(mu_tot * m_tot, G.astype(jnp.float32),
                                (((1,), (0,)), ((), ())),
                                preferred_element_type=jnp.float32)
        r_out_ref[...] = _COEF * d / jnp.maximum(ep_ref[...], 1.0)


def kernel(pos, batch, q_A, mu_A, m_A, v_A, E_pred):
    N = pos.shape[0]
    B, S = E_pred.shape
    S3 = 3 * S
    F = 3 * S3 + S + 4

    K = 1000 if N % 1000 == 0 else (500 if N % 500 == 0 else N)
    grid = N // K

    ids = batch.astype(jnp.int32).reshape(N, 1)
    mu_f = mu_A.reshape(N, S3)
    m_f = m_A.reshape(N, S3)
    v_f = v_A.reshape(N, S3)

    f32 = jnp.float32
    acc, mu_flat, m_flat, r = pl.pallas_call(
        _agg_kernel,
        grid=(grid,),
        in_specs=[
            pl.BlockSpec((K, 1), lambda i: (i, 0)),
            pl.BlockSpec((K, 3), lambda i: (i, 0)),
            pl.BlockSpec((K, S), lambda i: (i, 0)),
            pl.BlockSpec((K, S3), lambda i: (i, 0)),
            pl.BlockSpec((K, S3), lambda i: (i, 0)),
            pl.BlockSpec((K, S3), lambda i: (i, 0)),
            pl.BlockSpec((B, S), lambda i: (0, 0)),
        ],
        out_specs=[
            pl.BlockSpec((B, F), lambda i: (0, 0)),
            pl.BlockSpec((B, S3), lambda i: (0, 0)),
            pl.BlockSpec((B, S3), lambda i: (0, 0)),
            pl.BlockSpec((B, S), lambda i: (0, 0)),
        ],
        out_shape=[
            jax.ShapeDtypeStruct((B, F), f32),
            jax.ShapeDtypeStruct((B, S3), f32),
            jax.ShapeDtypeStruct((B, S3), f32),
            jax.ShapeDtypeStruct((B, S), f32),
        ],
    )(ids, pos, q_A, mu_f, m_f, v_f, E_pred)
    del acc
    return (mu_flat.reshape(B, S, 3), m_flat.reshape(B, S, 3), r)
